# baseline (speedup 1.0000x reference)
import functools

import jax
import jax.numpy as jnp
from jax import lax
from jax.experimental import pallas as pl
from jax.experimental.pallas import tpu as pltpu
from jax.experimental.pallas import tpu_sc as plsc

F = 26
V = 100000
E = 32
B = 4096
L = 50
T = B * F
N = T * L
NCOL = T * E

NC = 2
NS = 16
NW = NC * NS
SCHUNK = 640
NGAT = SCHUNK // 128
NSUP = N // (NW * SCHUNK)
WB = SCHUNK * E // 128
YROWS = N * E // 128

_mesh = plsc.VectorSubcoreMesh(core_axis_name="c", subcore_axis_name="s")


@functools.partial(
    pl.kernel,
    mesh=_mesh,
    compiler_params=pltpu.CompilerParams(use_tc_tiling_on_sc=False),
    out_type=jax.ShapeDtypeStruct((YROWS, 128), jnp.float32),
    scratch_types=[
        pltpu.VMEM((SCHUNK,), jnp.int32),
        pltpu.VMEM((SCHUNK,), jnp.int32),
        pltpu.VMEM((SCHUNK, E), jnp.float32),
        pltpu.VMEM((SCHUNK, E), jnp.float32),
        pltpu.VMEM((WB, 128), jnp.float32),
        pltpu.VMEM((WB, 128), jnp.float32),
        pltpu.SemaphoreType.DMA,
        pltpu.SemaphoreType.DMA,
        pltpu.SemaphoreType.DMA,
        pltpu.SemaphoreType.DMA,
        pltpu.SemaphoreType.DMA,
        pltpu.SemaphoreType.DMA,
    ],
)
def _sc_gather(tab_hbm, idx_hbm, y_hbm, ig0, ig1, rg0, rg1, rw0, rw1,
               si0, si1, sg0, sg1, sw0, sw1):
    wid = lax.axis_index("s") * NC + lax.axis_index("c")
    base = wid * NSUP
    igs = (ig0, ig1)
    rgs = (rg0, rg1)
    rws = (rw0, rw1)
    sis = (si0, si1)
    sgs = (sg0, sg1)
    sws = (sw0, sw1)

    def fire_gathers(ig, rg, sg):
        for j in range(NGAT):
            pltpu.async_copy(tab_hbm.at[ig.at[pl.ds(j * 128, 128)]],
                             rg.at[pl.ds(j * 128, 128)], sg)

    def drain_gathers(ig, rg, sg):
        for j in range(NGAT):
            pltpu.make_async_copy(tab_hbm.at[ig.at[pl.ds(j * 128, 128)]],
                                  rg.at[pl.ds(j * 128, 128)], sg).wait()

    pltpu.async_copy(idx_hbm.at[pl.ds(base * SCHUNK, SCHUNK)], ig0, si0)
    pltpu.async_copy(idx_hbm.at[pl.ds((base + 1) * SCHUNK, SCHUNK)], ig1, si1)
    pltpu.make_async_copy(idx_hbm.at[pl.ds(0, SCHUNK)], ig0, si0).wait()
    fire_gathers(ig0, rg0, sg0)

    @pl.loop(0, NSUP, step=2)
    def _(s0):
        for b in range(2):
            s = s0 + b
            o = 1 - b
            drain_gathers(igs[b], rgs[b], sgs[b])
            @pl.when(s0 < NSUP - 2)
            def _():
                pltpu.async_copy(
                    idx_hbm.at[pl.ds((base + s + 2) * SCHUNK, SCHUNK)],
                    igs[b], sis[b])
            if b == 0:
                pltpu.make_async_copy(
                    idx_hbm.at[pl.ds(0, SCHUNK)], igs[o], sis[o]).wait()
                fire_gathers(igs[o], rgs[o], sgs[o])
            else:
                @pl.when(s0 < NSUP - 2)
                def _():
                    pltpu.make_async_copy(
                        idx_hbm.at[pl.ds(0, SCHUNK)], igs[o], sis[o]).wait()
                    fire_gathers(igs[o], rgs[o], sgs[o])
            @pl.when(s0 > 0)
            def _():
                pltpu.make_async_copy(rws[b], y_hbm.at[pl.ds(0, WB)],
                                      sws[b]).wait()
            rg, rw = rgs[b], rws[b]

            @pl.loop(0, WB)
            def _(r):
                for c in range(8):
                    rw[r, pl.ds(c * 16, 16)] = (
                        rg[r * 4 + c // 2, pl.ds((c % 2) * 16, 16)])
            pltpu.async_copy(rw, y_hbm.at[pl.ds((base + s) * WB, WB)], sws[b])

    for b in range(2):
        pltpu.make_async_copy(rws[b], y_hbm.at[pl.ds(0, WB)], sws[b]).wait()


CB = 64
GRID = NCOL // (CB * 128)


def _tr_body(x_ref, o_ref):
    for c in range(CB):
        o_ref[pl.ds(c * 128, 128), :] = jnp.transpose(x_ref[:, c, :], (1, 0))


_tc_transpose = pl.pallas_call(
    _tr_body,
    grid=(GRID,),
    in_specs=[pl.BlockSpec((L, CB, 128), lambda i: (0, i, 0))],
    out_specs=pl.BlockSpec((CB * 128, L), lambda i: (i, 0)),
    out_shape=jax.ShapeDtypeStruct((NCOL, L), jnp.float32),
    compiler_params=pltpu.CompilerParams(dimension_semantics=("parallel",)),
)


@jax.jit
def kernel(inputs, tables):
    tab = tables.reshape(F * V, E)
    offs = (jnp.arange(F, dtype=jnp.int32) * V)[None, :, None]
    gidx = (inputs.astype(jnp.int32) + offs)
    gidx = gidx.transpose(2, 0, 1).reshape(N)
    y = _sc_gather(tab, gidx)
    y3 = y.reshape(L, NCOL // 128, 128)
    out = _tc_transpose(y3)
    return out.reshape(B, F * E, L)


# speedup vs baseline: 1.0090x; 1.0090x over previous
import functools

import jax
import jax.numpy as jnp
from jax import lax
from jax.experimental import pallas as pl
from jax.experimental.pallas import tpu as pltpu
from jax.experimental.pallas import tpu_sc as plsc

F = 26
V = 100000
E = 32
B = 4096
L = 50
T = B * F
N = T * L
NCOL = T * E

NC = 2
NS = 16
NW = NC * NS
SCHUNK = 640
NGAT = SCHUNK // 128
NSUP = N // (NW * SCHUNK)
WB = SCHUNK * E // 128
YROWS = N * E // 128

_mesh = plsc.VectorSubcoreMesh(core_axis_name="c", subcore_axis_name="s")


@functools.partial(
    pl.kernel,
    mesh=_mesh,
    compiler_params=pltpu.CompilerParams(use_tc_tiling_on_sc=False),
    out_type=jax.ShapeDtypeStruct((YROWS, 128), jnp.float32),
    scratch_types=[
        pltpu.VMEM((SCHUNK,), jnp.int32),
        pltpu.VMEM((SCHUNK,), jnp.int32),
        pltpu.VMEM((SCHUNK, E), jnp.float32),
        pltpu.VMEM((SCHUNK, E), jnp.float32),
        pltpu.VMEM((WB, 128), jnp.float32),
        pltpu.VMEM((WB, 128), jnp.float32),
        pltpu.SemaphoreType.DMA,
        pltpu.SemaphoreType.DMA,
        pltpu.SemaphoreType.DMA,
        pltpu.SemaphoreType.DMA,
        pltpu.SemaphoreType.DMA,
        pltpu.SemaphoreType.DMA,
    ],
)
def _sc_gather(tab_hbm, idx_hbm, y_hbm, ig0, ig1, rg0, rg1, rw0, rw1,
               si0, si1, sg0, sg1, sw0, sw1):
    wid = lax.axis_index("s") * NC + lax.axis_index("c")
    base = wid * NSUP
    igs = (ig0, ig1)
    rgs = (rg0, rg1)
    rws = (rw0, rw1)
    sis = (si0, si1)
    sgs = (sg0, sg1)
    sws = (sw0, sw1)

    def fire_gathers(ig, rg, sg):
        for j in range(NGAT):
            pltpu.async_copy(tab_hbm.at[ig.at[pl.ds(j * 128, 128)]],
                             rg.at[pl.ds(j * 128, 128)], sg)

    def drain_gathers(ig, rg, sg):
        for j in range(NGAT):
            pltpu.make_async_copy(tab_hbm.at[ig.at[pl.ds(j * 128, 128)]],
                                  rg.at[pl.ds(j * 128, 128)], sg).wait()

    pltpu.async_copy(idx_hbm.at[pl.ds(base * SCHUNK, SCHUNK)], ig0, si0)
    pltpu.async_copy(idx_hbm.at[pl.ds((base + 1) * SCHUNK, SCHUNK)], ig1, si1)
    pltpu.make_async_copy(idx_hbm.at[pl.ds(0, SCHUNK)], ig0, si0).wait()
    fire_gathers(ig0, rg0, sg0)

    @pl.loop(0, NSUP, step=2)
    def _(s0):
        for b in range(2):
            s = s0 + b
            o = 1 - b
            if b == 0:
                pltpu.make_async_copy(
                    idx_hbm.at[pl.ds(0, SCHUNK)], igs[o], sis[o]).wait()
                fire_gathers(igs[o], rgs[o], sgs[o])
            else:
                @pl.when(s0 < NSUP - 2)
                def _():
                    pltpu.make_async_copy(
                        idx_hbm.at[pl.ds(0, SCHUNK)], igs[o], sis[o]).wait()
                    fire_gathers(igs[o], rgs[o], sgs[o])
            drain_gathers(igs[b], rgs[b], sgs[b])
            @pl.when(s0 < NSUP - 2)
            def _():
                pltpu.async_copy(
                    idx_hbm.at[pl.ds((base + s + 2) * SCHUNK, SCHUNK)],
                    igs[b], sis[b])
            @pl.when(s0 > 0)
            def _():
                pltpu.make_async_copy(rws[b], y_hbm.at[pl.ds(0, WB)],
                                      sws[b]).wait()
            rg, rw = rgs[b], rws[b]

            @pl.loop(0, WB)
            def _(r):
                for c in range(8):
                    rw[r, pl.ds(c * 16, 16)] = (
                        rg[r * 4 + c // 2, pl.ds((c % 2) * 16, 16)])
            pltpu.async_copy(rw, y_hbm.at[pl.ds((base + s) * WB, WB)], sws[b])

    for b in range(2):
        pltpu.make_async_copy(rws[b], y_hbm.at[pl.ds(0, WB)], sws[b]).wait()


CB = 64
GRID = NCOL // (CB * 128)


def _tr_body(x_ref, o_ref):
    for c in range(CB):
        o_ref[pl.ds(c * 128, 128), :] = jnp.transpose(x_ref[:, c, :], (1, 0))


_tc_transpose = pl.pallas_call(
    _tr_body,
    grid=(GRID,),
    in_specs=[pl.BlockSpec((L, CB, 128), lambda i: (0, i, 0))],
    out_specs=pl.BlockSpec((CB * 128, L), lambda i: (i, 0)),
    out_shape=jax.ShapeDtypeStruct((NCOL, L), jnp.float32),
    compiler_params=pltpu.CompilerParams(dimension_semantics=("parallel",)),
)


@jax.jit
def kernel(inputs, tables):
    tab = tables.reshape(F * V, E)
    offs = (jnp.arange(F, dtype=jnp.int32) * V)[None, :, None]
    gidx = (inputs.astype(jnp.int32) + offs)
    gidx = gidx.transpose(2, 0, 1).reshape(N)
    y = _sc_gather(tab, gidx)
    y3 = y.reshape(L, NCOL // 128, 128)
    out = _tc_transpose(y3)
    return out.reshape(B, F * E, L)


# speedup vs baseline: 1.0129x; 1.0038x over previous
import functools

import jax
import jax.numpy as jnp
from jax import lax
from jax.experimental import pallas as pl
from jax.experimental.pallas import tpu as pltpu
from jax.experimental.pallas import tpu_sc as plsc

F = 26
V = 100000
E = 32
B = 4096
L = 50
T = B * F
N = T * L
NCOL = T * E

HB = B // 2
NH = N // 2
YROWS_H = NH * E // 128

NC = 2
NS = 16
NW = NC * NS
SCHUNK = 640
NGAT = SCHUNK // 128
NSUP = NH // (NW * SCHUNK)
WB = SCHUNK * E // 128

_mesh = plsc.VectorSubcoreMesh(core_axis_name="c", subcore_axis_name="s")


@functools.partial(
    pl.kernel,
    mesh=_mesh,
    compiler_params=pltpu.CompilerParams(use_tc_tiling_on_sc=False),
    out_type=jax.ShapeDtypeStruct((YROWS_H, 128), jnp.float32),
    scratch_types=[
        pltpu.VMEM((SCHUNK,), jnp.int32),
        pltpu.VMEM((SCHUNK,), jnp.int32),
        pltpu.VMEM((SCHUNK, E), jnp.float32),
        pltpu.VMEM((SCHUNK, E), jnp.float32),
        pltpu.VMEM((WB, 128), jnp.float32),
        pltpu.VMEM((WB, 128), jnp.float32),
        pltpu.SemaphoreType.DMA,
        pltpu.SemaphoreType.DMA,
        pltpu.SemaphoreType.DMA,
        pltpu.SemaphoreType.DMA,
        pltpu.SemaphoreType.DMA,
        pltpu.SemaphoreType.DMA,
    ],
)
def _sc_gather(tab_hbm, idx_hbm, y_hbm, ig0, ig1, rg0, rg1, rw0, rw1,
               si0, si1, sg0, sg1, sw0, sw1):
    wid = lax.axis_index("s") * NC + lax.axis_index("c")
    base = wid * NSUP
    igs = (ig0, ig1)
    rgs = (rg0, rg1)
    rws = (rw0, rw1)
    sis = (si0, si1)
    sgs = (sg0, sg1)
    sws = (sw0, sw1)

    def fire_gathers(ig, rg, sg):
        for j in range(NGAT):
            pltpu.async_copy(tab_hbm.at[ig.at[pl.ds(j * 128, 128)]],
                             rg.at[pl.ds(j * 128, 128)], sg)

    def drain_gathers(ig, rg, sg):
        for j in range(NGAT):
            pltpu.make_async_copy(tab_hbm.at[ig.at[pl.ds(j * 128, 128)]],
                                  rg.at[pl.ds(j * 128, 128)], sg).wait()

    pltpu.async_copy(idx_hbm.at[pl.ds(base * SCHUNK, SCHUNK)], ig0, si0)
    pltpu.async_copy(idx_hbm.at[pl.ds((base + 1) * SCHUNK, SCHUNK)], ig1, si1)
    pltpu.make_async_copy(idx_hbm.at[pl.ds(0, SCHUNK)], ig0, si0).wait()
    fire_gathers(ig0, rg0, sg0)

    @pl.loop(0, NSUP, step=2)
    def _(s0):
        for b in range(2):
            s = s0 + b
            o = 1 - b
            if b == 0:
                pltpu.make_async_copy(
                    idx_hbm.at[pl.ds(0, SCHUNK)], igs[o], sis[o]).wait()
                fire_gathers(igs[o], rgs[o], sgs[o])
            else:
                @pl.when(s0 < NSUP - 2)
                def _():
                    pltpu.make_async_copy(
                        idx_hbm.at[pl.ds(0, SCHUNK)], igs[o], sis[o]).wait()
                    fire_gathers(igs[o], rgs[o], sgs[o])
            drain_gathers(igs[b], rgs[b], sgs[b])
            @pl.when(s0 < NSUP - 2)
            def _():
                pltpu.async_copy(
                    idx_hbm.at[pl.ds((base + s + 2) * SCHUNK, SCHUNK)],
                    igs[b], sis[b])
            @pl.when(s0 > 0)
            def _():
                pltpu.make_async_copy(rws[b], y_hbm.at[pl.ds(0, WB)],
                                      sws[b]).wait()
            rg, rw = rgs[b], rws[b]

            @pl.loop(0, WB)
            def _(r):
                for c in range(8):
                    rw[r, pl.ds(c * 16, 16)] = (
                        rg[r * 4 + c // 2, pl.ds((c % 2) * 16, 16)])
            pltpu.async_copy(rw, y_hbm.at[pl.ds((base + s) * WB, WB)], sws[b])

    for b in range(2):
        pltpu.make_async_copy(rws[b], y_hbm.at[pl.ds(0, WB)], sws[b]).wait()


CB = 64
GRID_H = NCOL // 2 // (CB * 128)


def _tr_body(x_ref, o_ref):
    for c in range(CB):
        o_ref[pl.ds(c * 128, 128), :] = jnp.transpose(x_ref[:, c, :], (1, 0))


def _tr_body2(x_ref, prev_ref, o_ref):
    del prev_ref
    _tr_body(x_ref, o_ref)


_tc_transpose1 = pl.pallas_call(
    _tr_body,
    grid=(GRID_H,),
    in_specs=[pl.BlockSpec((L, CB, 128), lambda i: (0, i, 0))],
    out_specs=pl.BlockSpec((CB * 128, L), lambda i: (i, 0)),
    out_shape=jax.ShapeDtypeStruct((NCOL, L), jnp.float32),
)

_tc_transpose2 = pl.pallas_call(
    _tr_body2,
    grid=(GRID_H,),
    in_specs=[pl.BlockSpec((L, CB, 128), lambda i: (0, i, 0)),
              pl.BlockSpec(memory_space=pltpu.MemorySpace.HBM)],
    out_specs=pl.BlockSpec((CB * 128, L), lambda i: (GRID_H + i, 0)),
    out_shape=jax.ShapeDtypeStruct((NCOL, L), jnp.float32),
    input_output_aliases={1: 0},
)


@jax.jit
def kernel(inputs, tables):
    tab = tables.reshape(F * V, E)
    offs = (jnp.arange(F, dtype=jnp.int32) * V)[None, :, None]
    gidx = (inputs.astype(jnp.int32) + offs).transpose(2, 0, 1)
    g1 = gidx[:, :HB, :].reshape(NH)
    g2 = gidx[:, HB:, :].reshape(NH)
    y1 = _sc_gather(tab, g1)
    y2 = _sc_gather(tab, g2)
    o1 = _tc_transpose1(y1.reshape(L, NCOL // 2 // 128, 128))
    out = _tc_transpose2(y2.reshape(L, NCOL // 2 // 128, 128), o1)
    return out.reshape(B, F * E, L)


# speedup vs baseline: 1.0360x; 1.0229x over previous
import functools

import jax
import jax.numpy as jnp
from jax import lax
from jax.experimental import pallas as pl
from jax.experimental.pallas import tpu as pltpu
from jax.experimental.pallas import tpu_sc as plsc

F = 26
V = 100000
E = 32
B = 4096
L = 50
T = B * F
N = T * L
NCOL = T * E

HB = B // 2
NH = N // 2
YROWS_H = NH * E // 128

NC = 2
NS = 16
NW = NC * NS
SCHUNK = 640
NGAT = SCHUNK // 128
NSUP = NH // (NW * SCHUNK)
WB = SCHUNK * E // 128

_mesh = plsc.VectorSubcoreMesh(core_axis_name="c", subcore_axis_name="s")


@functools.partial(
    pl.kernel,
    mesh=_mesh,
    compiler_params=pltpu.CompilerParams(use_tc_tiling_on_sc=False),
    out_type=jax.ShapeDtypeStruct((YROWS_H, 128), jnp.float32),
    scratch_types=[
        pltpu.VMEM((SCHUNK,), jnp.int32),
        pltpu.VMEM((SCHUNK,), jnp.int32),
        pltpu.VMEM((SCHUNK, E), jnp.float32),
        pltpu.VMEM((SCHUNK, E), jnp.float32),
        pltpu.VMEM((WB, 128), jnp.float32),
        pltpu.VMEM((WB, 128), jnp.float32),
        pltpu.SemaphoreType.DMA,
        pltpu.SemaphoreType.DMA,
        pltpu.SemaphoreType.DMA,
        pltpu.SemaphoreType.DMA,
        pltpu.SemaphoreType.DMA,
        pltpu.SemaphoreType.DMA,
    ],
)
def _sc_gather(tab_hbm, idx_hbm, y_hbm, ig0, ig1, rg0, rg1, rw0, rw1,
               si0, si1, sg0, sg1, sw0, sw1):
    wid = lax.axis_index("s") * NC + lax.axis_index("c")
    base = wid * NSUP
    igs = (ig0, ig1)
    rgs = (rg0, rg1)
    rws = (rw0, rw1)
    sis = (si0, si1)
    sgs = (sg0, sg1)
    sws = (sw0, sw1)

    def fire_gathers(ig, rg, sg):
        for j in range(NGAT):
            pltpu.async_copy(tab_hbm.at[ig.at[pl.ds(j * 128, 128)]],
                             rg.at[pl.ds(j * 128, 128)], sg)

    def drain_gathers(ig, rg, sg):
        for j in range(NGAT):
            pltpu.make_async_copy(tab_hbm.at[ig.at[pl.ds(j * 128, 128)]],
                                  rg.at[pl.ds(j * 128, 128)], sg).wait()

    pltpu.async_copy(idx_hbm.at[pl.ds(base * SCHUNK, SCHUNK)], ig0, si0)
    pltpu.async_copy(idx_hbm.at[pl.ds((base + 1) * SCHUNK, SCHUNK)], ig1, si1)
    pltpu.make_async_copy(idx_hbm.at[pl.ds(0, SCHUNK)], ig0, si0).wait()
    fire_gathers(ig0, rg0, sg0)

    @pl.loop(0, NSUP, step=2)
    def _(s0):
        for b in range(2):
            s = s0 + b
            o = 1 - b
            if b == 0:
                pltpu.make_async_copy(
                    idx_hbm.at[pl.ds(0, SCHUNK)], igs[o], sis[o]).wait()
                fire_gathers(igs[o], rgs[o], sgs[o])
            else:
                @pl.when(s0 < NSUP - 2)
                def _():
                    pltpu.make_async_copy(
                        idx_hbm.at[pl.ds(0, SCHUNK)], igs[o], sis[o]).wait()
                    fire_gathers(igs[o], rgs[o], sgs[o])
            drain_gathers(igs[b], rgs[b], sgs[b])
            @pl.when(s0 < NSUP - 2)
            def _():
                pltpu.async_copy(
                    idx_hbm.at[pl.ds((base + s + 2) * SCHUNK, SCHUNK)],
                    igs[b], sis[b])
            @pl.when(s0 > 0)
            def _():
                pltpu.make_async_copy(rws[b], y_hbm.at[pl.ds(0, WB)],
                                      sws[b]).wait()
            rg, rw = rgs[b], rws[b]

            @pl.loop(0, WB)
            def _(r):
                for c in range(8):
                    rw[r, pl.ds(c * 16, 16)] = (
                        rg[r * 4 + c // 2, pl.ds((c % 2) * 16, 16)])
            pltpu.async_copy(rw, y_hbm.at[pl.ds((base + s) * WB, WB)], sws[b])

    for b in range(2):
        pltpu.make_async_copy(rws[b], y_hbm.at[pl.ds(0, WB)], sws[b]).wait()


CB = 128
GRID_H = NCOL // 2 // (CB * 128)


def _tr_body(x_ref, o_ref):
    for c in range(CB):
        o_ref[pl.ds(c * 128, 128), :] = jnp.transpose(x_ref[:, c, :], (1, 0))


def _tr_body2(x_ref, prev_ref, o_ref):
    del prev_ref
    _tr_body(x_ref, o_ref)


_tc_transpose1 = pl.pallas_call(
    _tr_body,
    grid=(GRID_H,),
    in_specs=[pl.BlockSpec((L, CB, 128), lambda i: (0, i, 0))],
    out_specs=pl.BlockSpec((CB * 128, L), lambda i: (i, 0)),
    out_shape=jax.ShapeDtypeStruct((NCOL, L), jnp.float32),
)

_tc_transpose2 = pl.pallas_call(
    _tr_body2,
    grid=(GRID_H,),
    in_specs=[pl.BlockSpec((L, CB, 128), lambda i: (0, i, 0)),
              pl.BlockSpec(memory_space=pltpu.MemorySpace.HBM)],
    out_specs=pl.BlockSpec((CB * 128, L), lambda i: (GRID_H + i, 0)),
    out_shape=jax.ShapeDtypeStruct((NCOL, L), jnp.float32),
    input_output_aliases={1: 0},
)


@jax.jit
def kernel(inputs, tables):
    tab = tables.reshape(F * V, E)
    offs = (jnp.arange(F, dtype=jnp.int32) * V)[None, :, None]
    gidx = (inputs.astype(jnp.int32) + offs).transpose(2, 0, 1)
    g1 = gidx[:, :HB, :].reshape(NH)
    g2 = gidx[:, HB:, :].reshape(NH)
    y1 = _sc_gather(tab, g1)
    y2 = _sc_gather(tab, g2)
    o1 = _tc_transpose1(y1.reshape(L, NCOL // 2 // 128, 128))
    out = _tc_transpose2(y2.reshape(L, NCOL // 2 // 128, 128), o1)
    return out.reshape(B, F * E, L)
